# Initial kernel scaffold; baseline (speedup 1.0000x reference)
import jax
import jax.numpy as jnp
from jax.experimental import pallas as pl


def kernel(phoneme_indices, phoneme_table, sutra_table, position_table, sutra_lookup, position_lookup, W, b):
    raise NotImplementedError("write your pallas kernel here")


# speedup vs baseline: 20.9407x; 20.9407x over previous
import functools

import jax
import jax.numpy as jnp
from jax import lax
from jax.experimental import pallas as pl
from jax.experimental.pallas import tpu as pltpu
from jax.experimental.pallas import tpu_sc as plsc

B, S, V, D = 1024, 200, 100000, 128
N = B * S

NC, NS = 2, 16
NW = NC * NS
NPW = N // NW
CHUNK = 128
NCHUNK = NPW // CHUNK
NBUF = 2

TBLK = 2048


def _fused_table_body(pt_ref, slk_ref, plk_ref, st_ref, po_ref, w_ref, b_ref,
                      out_ref):
    w0 = w_ref[0:D, :]
    w1 = w_ref[D:2 * D, :]
    w2 = w_ref[2 * D:3 * D, :]
    sproj = jnp.dot(st_ref[...], w1, preferred_element_type=jnp.float32)
    pproj = jnp.dot(po_ref[...], w2, preferred_element_type=jnp.float32)
    main = jnp.dot(pt_ref[...], w0, preferred_element_type=jnp.float32)
    iot = lax.broadcasted_iota(jnp.int32, (TBLK, 16), 1)
    oh_s = (slk_ref[...] == iot).astype(jnp.float32)
    oh_p = (plk_ref[...] == iot).astype(jnp.float32)
    out_ref[...] = (main
                    + jnp.dot(oh_s, sproj, preferred_element_type=jnp.float32)
                    + jnp.dot(oh_p, pproj, preferred_element_type=jnp.float32)
                    + b_ref[...])


def _build_fused_table(phoneme_table, sutra_table, position_table,
                       sutra_lookup, position_lookup, W, b):
    st16 = jnp.zeros((16, D), jnp.float32).at[:15].set(sutra_table)
    po16 = jnp.zeros((16, D), jnp.float32).at[:11].set(position_table)
    grid = pl.cdiv(V, TBLK)
    return pl.pallas_call(
        _fused_table_body,
        grid=(grid,),
        in_specs=[
            pl.BlockSpec((TBLK, D), lambda i: (i, 0)),
            pl.BlockSpec((TBLK, 1), lambda i: (i, 0)),
            pl.BlockSpec((TBLK, 1), lambda i: (i, 0)),
            pl.BlockSpec((16, D), lambda i: (0, 0)),
            pl.BlockSpec((16, D), lambda i: (0, 0)),
            pl.BlockSpec((3 * D, D), lambda i: (0, 0)),
            pl.BlockSpec((1, D), lambda i: (0, 0)),
        ],
        out_specs=pl.BlockSpec((TBLK, D), lambda i: (i, 0)),
        out_shape=jax.ShapeDtypeStruct((V, D), jnp.float32),
    )(phoneme_table, sutra_lookup[:, None], position_lookup[:, None],
      st16, po16, W, b[None, :])


def _gather_rows(table, idx):
    mesh = plsc.VectorSubcoreMesh(core_axis_name="c", subcore_axis_name="s")

    @functools.partial(
        pl.kernel,
        out_type=jax.ShapeDtypeStruct((N, D), jnp.float32),
        mesh=mesh,
        scratch_types=[
            pltpu.VMEM((NCHUNK, CHUNK), jnp.int32),
            pltpu.VMEM((NBUF, CHUNK, D), jnp.float32),
            pltpu.SemaphoreType.DMA,
            pltpu.SemaphoreType.DMA,
        ],
    )
    def gather_kernel(table_hbm, idx_hbm, out_hbm, idx_v, rows_v, sem0, sem1):
        wid = lax.axis_index("s") * NC + lax.axis_index("c")
        base = wid * NPW
        pltpu.sync_copy(idx_hbm.at[wid], idx_v)
        sems = [sem0, sem1]
        for bi in range(NBUF):
            pltpu.async_copy(table_hbm.at[idx_v.at[bi]], rows_v.at[bi],
                             sems[bi])

        def steady(j0):
            for bi in range(NBUF):
                j = j0 + bi
                pltpu.make_async_copy(table_hbm.at[idx_v.at[j]],
                                      rows_v.at[bi], sems[bi]).wait()
                pltpu.sync_copy(rows_v.at[bi],
                                out_hbm.at[pl.ds(base + j * CHUNK, CHUNK)])
                pltpu.async_copy(table_hbm.at[idx_v.at[j + NBUF]],
                                 rows_v.at[bi], sems[bi])

        pl.loop(0, NCHUNK - NBUF, step=NBUF)(steady)
        for bi in range(NBUF):
            j = NCHUNK - NBUF + bi
            pltpu.make_async_copy(table_hbm.at[idx_v.at[j]], rows_v.at[bi],
                                  sems[bi]).wait()
            pltpu.sync_copy(rows_v.at[bi],
                            out_hbm.at[pl.ds(base + j * CHUNK, CHUNK)])

    return gather_kernel(table, idx)


def kernel(phoneme_indices, phoneme_table, sutra_table, position_table,
           sutra_lookup, position_lookup, W, b):
    fused = _build_fused_table(phoneme_table, sutra_table, position_table,
                               sutra_lookup, position_lookup, W, b)
    idx = phoneme_indices.reshape(NW, NCHUNK, CHUNK).astype(jnp.int32)
    out = _gather_rows(fused, idx)
    return out.reshape(B, S, D)


# speedup vs baseline: 33.6504x; 1.6069x over previous
import functools

import jax
import jax.numpy as jnp
from jax import lax
from jax.experimental import pallas as pl
from jax.experimental.pallas import tpu as pltpu
from jax.experimental.pallas import tpu_sc as plsc

B, S, V, D = 1024, 200, 100000, 128
N = B * S

NC, NS = 2, 16
NW = NC * NS
NPW = N // NW
CHUNK = 128
NCHUNK = NPW // CHUNK
NBUF = 2

TBLK = 2048


def _fused_table_body(pt_ref, slk_ref, plk_ref, st_ref, po_ref, w_ref, b_ref,
                      out_ref):
    w0 = w_ref[0:D, :]
    w1 = w_ref[D:2 * D, :]
    w2 = w_ref[2 * D:3 * D, :]
    sproj = jnp.dot(st_ref[...], w1, preferred_element_type=jnp.float32)
    pproj = jnp.dot(po_ref[...], w2, preferred_element_type=jnp.float32)
    main = jnp.dot(pt_ref[...], w0, preferred_element_type=jnp.float32)
    iot = lax.broadcasted_iota(jnp.int32, (TBLK, 16), 1)
    oh_s = (slk_ref[...][:, None] == iot).astype(jnp.float32)
    oh_p = (plk_ref[...][:, None] == iot).astype(jnp.float32)
    out_ref[...] = (main
                    + jnp.dot(oh_s, sproj, preferred_element_type=jnp.float32)
                    + jnp.dot(oh_p, pproj, preferred_element_type=jnp.float32)
                    + b_ref[...])


def _build_fused_table(phoneme_table, sutra_table, position_table,
                       sutra_lookup, position_lookup, W, b):
    st16 = jnp.zeros((16, D), jnp.float32).at[:15].set(sutra_table)
    po16 = jnp.zeros((16, D), jnp.float32).at[:11].set(position_table)
    grid = pl.cdiv(V, TBLK)
    return pl.pallas_call(
        _fused_table_body,
        grid=(grid,),
        in_specs=[
            pl.BlockSpec((TBLK, D), lambda i: (i, 0)),
            pl.BlockSpec((TBLK,), lambda i: (i,)),
            pl.BlockSpec((TBLK,), lambda i: (i,)),
            pl.BlockSpec((16, D), lambda i: (0, 0)),
            pl.BlockSpec((16, D), lambda i: (0, 0)),
            pl.BlockSpec((3 * D, D), lambda i: (0, 0)),
            pl.BlockSpec((1, D), lambda i: (0, 0)),
        ],
        out_specs=pl.BlockSpec((TBLK, D), lambda i: (i, 0)),
        out_shape=jax.ShapeDtypeStruct((V, D), jnp.float32),
    )(phoneme_table, sutra_lookup, position_lookup,
      st16, po16, W, b[None, :])


def _gather_rows(table, idx):
    mesh = plsc.VectorSubcoreMesh(core_axis_name="c", subcore_axis_name="s")

    @functools.partial(
        pl.kernel,
        out_type=jax.ShapeDtypeStruct((N, D), jnp.float32),
        mesh=mesh,
        scratch_types=[
            pltpu.VMEM((NCHUNK, CHUNK), jnp.int32),
            pltpu.VMEM((NBUF, CHUNK, D), jnp.float32),
            pltpu.SemaphoreType.DMA,
            pltpu.SemaphoreType.DMA,
        ],
    )
    def gather_kernel(table_hbm, idx_hbm, out_hbm, idx_v, rows_v, sem0, sem1):
        wid = lax.axis_index("s") * NC + lax.axis_index("c")
        base = wid * NPW
        pltpu.sync_copy(idx_hbm.at[wid], idx_v)
        sems = [sem0, sem1]
        for bi in range(NBUF):
            pltpu.async_copy(table_hbm.at[idx_v.at[bi]], rows_v.at[bi],
                             sems[bi])

        def steady(j0):
            for bi in range(NBUF):
                j = j0 + bi
                pltpu.make_async_copy(table_hbm.at[idx_v.at[j]],
                                      rows_v.at[bi], sems[bi]).wait()
                pltpu.sync_copy(rows_v.at[bi],
                                out_hbm.at[pl.ds(base + j * CHUNK, CHUNK)])
                pltpu.async_copy(table_hbm.at[idx_v.at[j + NBUF]],
                                 rows_v.at[bi], sems[bi])

        pl.loop(0, NCHUNK - NBUF, step=NBUF)(steady)
        for bi in range(NBUF):
            j = NCHUNK - NBUF + bi
            pltpu.make_async_copy(table_hbm.at[idx_v.at[j]], rows_v.at[bi],
                                  sems[bi]).wait()
            pltpu.sync_copy(rows_v.at[bi],
                            out_hbm.at[pl.ds(base + j * CHUNK, CHUNK)])

    return gather_kernel(table, idx)


def kernel(phoneme_indices, phoneme_table, sutra_table, position_table,
           sutra_lookup, position_lookup, W, b):
    fused = _build_fused_table(phoneme_table, sutra_table, position_table,
                               sutra_lookup, position_lookup, W, b)
    idx = phoneme_indices.reshape(NW, NCHUNK, CHUNK).astype(jnp.int32)
    out = _gather_rows(fused, idx)
    return out.reshape(B, S, D)


# speedup vs baseline: 34.0663x; 1.0124x over previous
import functools

import jax
import jax.numpy as jnp
from jax import lax
from jax.experimental import pallas as pl
from jax.experimental.pallas import tpu as pltpu
from jax.experimental.pallas import tpu_sc as plsc

B, S, V, D = 1024, 200, 100000, 128
N = B * S

NC, NS = 2, 16
NW = NC * NS
NPW = N // NW
CHUNK = 128
NCHUNK = NPW // CHUNK
NBUF = 5

TBLK = 2048


def _fused_table_body(pt_ref, slk_ref, plk_ref, st_ref, po_ref, w_ref, b_ref,
                      out_ref):
    w0 = w_ref[0:D, :]
    w1 = w_ref[D:2 * D, :]
    w2 = w_ref[2 * D:3 * D, :]
    sproj = jnp.dot(st_ref[...], w1, preferred_element_type=jnp.float32)
    pproj = jnp.dot(po_ref[...], w2, preferred_element_type=jnp.float32)
    main = jnp.dot(pt_ref[...], w0, preferred_element_type=jnp.float32)
    iot = lax.broadcasted_iota(jnp.int32, (TBLK, 16), 1)
    oh_s = (slk_ref[...][:, None] == iot).astype(jnp.float32)
    oh_p = (plk_ref[...][:, None] == iot).astype(jnp.float32)
    out_ref[...] = (main
                    + jnp.dot(oh_s, sproj, preferred_element_type=jnp.float32)
                    + jnp.dot(oh_p, pproj, preferred_element_type=jnp.float32)
                    + b_ref[...])


def _build_fused_table(phoneme_table, sutra_table, position_table,
                       sutra_lookup, position_lookup, W, b):
    st16 = jnp.zeros((16, D), jnp.float32).at[:15].set(sutra_table)
    po16 = jnp.zeros((16, D), jnp.float32).at[:11].set(position_table)
    grid = pl.cdiv(V, TBLK)
    return pl.pallas_call(
        _fused_table_body,
        grid=(grid,),
        in_specs=[
            pl.BlockSpec((TBLK, D), lambda i: (i, 0)),
            pl.BlockSpec((TBLK,), lambda i: (i,)),
            pl.BlockSpec((TBLK,), lambda i: (i,)),
            pl.BlockSpec((16, D), lambda i: (0, 0)),
            pl.BlockSpec((16, D), lambda i: (0, 0)),
            pl.BlockSpec((3 * D, D), lambda i: (0, 0)),
            pl.BlockSpec((1, D), lambda i: (0, 0)),
        ],
        out_specs=pl.BlockSpec((TBLK, D), lambda i: (i, 0)),
        out_shape=jax.ShapeDtypeStruct((V, D), jnp.float32),
    )(phoneme_table, sutra_lookup, position_lookup,
      st16, po16, W, b[None, :])


def _gather_rows(table, idx):
    mesh = plsc.VectorSubcoreMesh(core_axis_name="c", subcore_axis_name="s")

    @functools.partial(
        pl.kernel,
        out_type=jax.ShapeDtypeStruct((N, D), jnp.float32),
        mesh=mesh,
        scratch_types=[
            pltpu.VMEM((NCHUNK, CHUNK), jnp.int32),
            pltpu.VMEM((NBUF, CHUNK, D), jnp.float32),
        ] + [pltpu.SemaphoreType.DMA] * NBUF,
    )
    def gather_kernel(table_hbm, idx_hbm, out_hbm, idx_v, rows_v, *sems):
        wid = lax.axis_index("s") * NC + lax.axis_index("c")
        base = wid * NPW
        pltpu.sync_copy(idx_hbm.at[wid], idx_v)
        for bi in range(NBUF):
            pltpu.async_copy(table_hbm.at[idx_v.at[bi]], rows_v.at[bi],
                             sems[bi])

        def steady(j0):
            for bi in range(NBUF):
                j = j0 + bi
                pltpu.make_async_copy(table_hbm.at[idx_v.at[j]],
                                      rows_v.at[bi], sems[bi]).wait()
                pltpu.sync_copy(rows_v.at[bi],
                                out_hbm.at[pl.ds(base + j * CHUNK, CHUNK)])
                pltpu.async_copy(table_hbm.at[idx_v.at[j + NBUF]],
                                 rows_v.at[bi], sems[bi])

        pl.loop(0, NCHUNK - NBUF, step=NBUF)(steady)
        for bi in range(NBUF):
            j = NCHUNK - NBUF + bi
            pltpu.make_async_copy(table_hbm.at[idx_v.at[j]], rows_v.at[bi],
                                  sems[bi]).wait()
            pltpu.sync_copy(rows_v.at[bi],
                            out_hbm.at[pl.ds(base + j * CHUNK, CHUNK)])

    return gather_kernel(table, idx)


def kernel(phoneme_indices, phoneme_table, sutra_table, position_table,
           sutra_lookup, position_lookup, W, b):
    fused = _build_fused_table(phoneme_table, sutra_table, position_table,
                               sutra_lookup, position_lookup, W, b)
    idx = phoneme_indices.reshape(NW, NCHUNK, CHUNK).astype(jnp.int32)
    out = _gather_rows(fused, idx)
    return out.reshape(B, S, D)


# speedup vs baseline: 38.1588x; 1.1201x over previous
import functools

import jax
import jax.numpy as jnp
from jax import lax
from jax.experimental import pallas as pl
from jax.experimental.pallas import tpu as pltpu
from jax.experimental.pallas import tpu_sc as plsc

B, S, V, D = 1024, 200, 100000, 128
N = B * S

NC, NS = 2, 16
NW = NC * NS
NPW = N // NW
CHUNK = 128
NCHUNK = NPW // CHUNK
NBUF = 5

TBLK = 4096


def _fused_table_body(pt_ref, slk_ref, plk_ref, st_ref, po_ref, w_ref, b_ref,
                      out_ref):
    w0 = w_ref[0:D, :]
    w1 = w_ref[D:2 * D, :]
    w2 = w_ref[2 * D:3 * D, :]
    sproj = jnp.dot(st_ref[...], w1, preferred_element_type=jnp.float32)
    pproj = jnp.dot(po_ref[...], w2, preferred_element_type=jnp.float32)
    main = jnp.dot(pt_ref[...], w0, preferred_element_type=jnp.float32)
    iot_s = lax.broadcasted_iota(jnp.int32, (TBLK, 15), 1)
    iot_p = lax.broadcasted_iota(jnp.int32, (TBLK, 11), 1)
    oh_s = (slk_ref[...][:, None] == iot_s).astype(jnp.float32)
    oh_p = (plk_ref[...][:, None] == iot_p).astype(jnp.float32)
    out_ref[...] = (main
                    + jnp.dot(oh_s, sproj, preferred_element_type=jnp.float32)
                    + jnp.dot(oh_p, pproj, preferred_element_type=jnp.float32)
                    + b_ref[...])


def _build_fused_table(phoneme_table, sutra_table, position_table,
                       sutra_lookup, position_lookup, W, b):
    grid = pl.cdiv(V, TBLK)
    return pl.pallas_call(
        _fused_table_body,
        grid=(grid,),
        in_specs=[
            pl.BlockSpec((TBLK, D), lambda i: (i, 0)),
            pl.BlockSpec((TBLK,), lambda i: (i,)),
            pl.BlockSpec((TBLK,), lambda i: (i,)),
            pl.BlockSpec((15, D), lambda i: (0, 0)),
            pl.BlockSpec((11, D), lambda i: (0, 0)),
            pl.BlockSpec((3 * D, D), lambda i: (0, 0)),
            pl.BlockSpec((1, D), lambda i: (0, 0)),
        ],
        out_specs=pl.BlockSpec((TBLK, D), lambda i: (i, 0)),
        out_shape=jax.ShapeDtypeStruct((V, D), jnp.float32),
    )(phoneme_table, sutra_lookup, position_lookup,
      sutra_table, position_table, W, b[None, :])


def _gather_rows(table, idx):
    mesh = plsc.VectorSubcoreMesh(core_axis_name="c", subcore_axis_name="s")

    @functools.partial(
        pl.kernel,
        out_type=jax.ShapeDtypeStruct((N, D), jnp.float32),
        mesh=mesh,
        scratch_types=[
            pltpu.VMEM((NCHUNK, CHUNK), jnp.int32),
            pltpu.VMEM((NBUF, CHUNK, D), jnp.float32),
        ] + [pltpu.SemaphoreType.DMA] * NBUF,
    )
    def gather_kernel(table_hbm, idx_hbm, out_hbm, idx_v, rows_v, *sems):
        wid = lax.axis_index("s") * NC + lax.axis_index("c")
        base = wid * NPW
        pltpu.sync_copy(idx_hbm.at[wid], idx_v)
        for bi in range(NBUF):
            pltpu.async_copy(table_hbm.at[idx_v.at[bi]], rows_v.at[bi],
                             sems[bi])

        def steady(j0):
            for bi in range(NBUF):
                j = j0 + bi
                pltpu.make_async_copy(table_hbm.at[idx_v.at[j]],
                                      rows_v.at[bi], sems[bi]).wait()
                pltpu.sync_copy(rows_v.at[bi],
                                out_hbm.at[pl.ds(base + j * CHUNK, CHUNK)])
                pltpu.async_copy(table_hbm.at[idx_v.at[j + NBUF]],
                                 rows_v.at[bi], sems[bi])

        pl.loop(0, NCHUNK - NBUF, step=NBUF)(steady)
        for bi in range(NBUF):
            j = NCHUNK - NBUF + bi
            pltpu.make_async_copy(table_hbm.at[idx_v.at[j]], rows_v.at[bi],
                                  sems[bi]).wait()
            pltpu.sync_copy(rows_v.at[bi],
                            out_hbm.at[pl.ds(base + j * CHUNK, CHUNK)])

    return gather_kernel(table, idx)


def kernel(phoneme_indices, phoneme_table, sutra_table, position_table,
           sutra_lookup, position_lookup, W, b):
    fused = _build_fused_table(phoneme_table, sutra_table, position_table,
                               sutra_lookup, position_lookup, W, b)
    idx = phoneme_indices.reshape(NW, NCHUNK, CHUNK).astype(jnp.int32)
    out = _gather_rows(fused, idx)
    return out.reshape(B, S, D)


# speedup vs baseline: 38.2463x; 1.0023x over previous
import functools

import jax
import jax.numpy as jnp
from jax import lax
from jax.experimental import pallas as pl
from jax.experimental.pallas import tpu as pltpu
from jax.experimental.pallas import tpu_sc as plsc

B, S, V, D = 1024, 200, 100000, 128
N = B * S

NC, NS = 2, 16
NW = NC * NS
NPW = N // NW
CHUNK = 128
NCHUNK = NPW // CHUNK
NBUF = 5

TBLK = 4096


def _fused_table_body(pt_ref, slk_ref, plk_ref, st_ref, po_ref, w_ref, b_ref,
                      out_ref):
    w0 = w_ref[0:D, :]
    w1 = w_ref[D:2 * D, :]
    w2 = w_ref[2 * D:3 * D, :]
    sproj = jnp.dot(st_ref[...], w1, preferred_element_type=jnp.float32)
    pproj = jnp.dot(po_ref[...], w2, preferred_element_type=jnp.float32)
    main = jnp.dot(pt_ref[...], w0, preferred_element_type=jnp.float32)
    iot_s = lax.broadcasted_iota(jnp.int32, (TBLK, 15), 1)
    iot_p = lax.broadcasted_iota(jnp.int32, (TBLK, 11), 1)
    oh_s = (slk_ref[...][:, None] == iot_s).astype(jnp.float32)
    oh_p = (plk_ref[...][:, None] == iot_p).astype(jnp.float32)
    out_ref[...] = (main
                    + jnp.dot(oh_s, sproj, preferred_element_type=jnp.float32)
                    + jnp.dot(oh_p, pproj, preferred_element_type=jnp.float32)
                    + b_ref[...])


def _build_fused_table(phoneme_table, sutra_table, position_table,
                       sutra_lookup, position_lookup, W, b):
    grid = pl.cdiv(V, TBLK)
    return pl.pallas_call(
        _fused_table_body,
        grid=(grid,),
        in_specs=[
            pl.BlockSpec((TBLK, D), lambda i: (i, 0)),
            pl.BlockSpec((TBLK,), lambda i: (i,)),
            pl.BlockSpec((TBLK,), lambda i: (i,)),
            pl.BlockSpec((15, D), lambda i: (0, 0)),
            pl.BlockSpec((11, D), lambda i: (0, 0)),
            pl.BlockSpec((3 * D, D), lambda i: (0, 0)),
            pl.BlockSpec((1, D), lambda i: (0, 0)),
        ],
        out_specs=pl.BlockSpec((TBLK, D), lambda i: (i, 0)),
        out_shape=jax.ShapeDtypeStruct((V, D), jnp.float32),
    )(phoneme_table, sutra_lookup, position_lookup,
      sutra_table, position_table, W, b[None, :])


def _gather_rows(table, idx):
    mesh = plsc.VectorSubcoreMesh(core_axis_name="c", subcore_axis_name="s")

    @functools.partial(
        pl.kernel,
        out_type=jax.ShapeDtypeStruct((N, D), jnp.float32),
        mesh=mesh,
        scratch_types=[
            pltpu.VMEM((NCHUNK, CHUNK), jnp.int32),
            pltpu.VMEM((NBUF, CHUNK, D), jnp.float32),
        ] + [pltpu.SemaphoreType.DMA] * (2 * NBUF),
    )
    def gather_kernel(table_hbm, idx_hbm, out_hbm, idx_v, rows_v, *sems):
        gsem, osem = sems[:NBUF], sems[NBUF:]
        wid = lax.axis_index("s") * NC + lax.axis_index("c")
        base = wid * NPW
        pltpu.sync_copy(idx_hbm.at[wid], idx_v)

        def gather_start(j, bi):
            pltpu.async_copy(table_hbm.at[idx_v.at[j]], rows_v.at[bi],
                             gsem[bi])

        def gather_wait(j, bi):
            pltpu.make_async_copy(table_hbm.at[idx_v.at[j]], rows_v.at[bi],
                                  gsem[bi]).wait()

        def out_start(j, bi):
            pltpu.async_copy(rows_v.at[bi],
                             out_hbm.at[pl.ds(base + j * CHUNK, CHUNK)],
                             osem[bi])

        def out_wait(j, bi):
            pltpu.make_async_copy(
                rows_v.at[bi], out_hbm.at[pl.ds(base + j * CHUNK, CHUNK)],
                osem[bi]).wait()

        for bi in range(NBUF):
            gather_start(bi, bi)
        gather_wait(0, 0)
        out_start(0, 0)

        def steady(j0):
            for dj in range(NBUF):
                j = j0 + dj
                nb = dj
                out_wait(j - 1, nb)
                gather_start(j + NBUF - 1, nb)
                bi = (dj + 1) % NBUF
                gather_wait(j, bi)
                out_start(j, bi)

        pl.loop(1, NCHUNK - NBUF + 1, step=NBUF)(steady)
        for j in range(NCHUNK - NBUF + 1, NCHUNK):
            bi = j % NBUF
            gather_wait(j, bi)
            out_start(j, bi)
        for j in range(NCHUNK - NBUF, NCHUNK):
            out_wait(j, j % NBUF)

    return gather_kernel(table, idx)


def kernel(phoneme_indices, phoneme_table, sutra_table, position_table,
           sutra_lookup, position_lookup, W, b):
    fused = _build_fused_table(phoneme_table, sutra_table, position_table,
                               sutra_lookup, position_lookup, W, b)
    idx = phoneme_indices.reshape(NW, NCHUNK, CHUNK).astype(jnp.int32)
    out = _gather_rows(fused, idx)
    return out.reshape(B, S, D)


# speedup vs baseline: 40.2434x; 1.0522x over previous
import functools

import jax
import jax.numpy as jnp
from jax import lax
from jax.experimental import pallas as pl
from jax.experimental.pallas import tpu as pltpu
from jax.experimental.pallas import tpu_sc as plsc

B, S, V, D = 1024, 200, 100000, 128
N = B * S

NC, NS = 2, 16
NW = NC * NS
NPW = N // NW
CHUNK = 128
NCHUNK = NPW // CHUNK
NBUF = 5

TBLK = 8192


def _fused_table_body(pt_ref, slk_ref, plk_ref, st_ref, po_ref, w_ref, b_ref,
                      out_ref):
    w0 = w_ref[0:D, :]
    w1 = w_ref[D:2 * D, :]
    w2 = w_ref[2 * D:3 * D, :]
    sproj = jnp.dot(st_ref[...], w1, preferred_element_type=jnp.float32)
    pproj = jnp.dot(po_ref[...], w2, preferred_element_type=jnp.float32)
    main = jnp.dot(pt_ref[...], w0, preferred_element_type=jnp.float32)
    iot_s = lax.broadcasted_iota(jnp.int32, (TBLK, 15), 1)
    iot_p = lax.broadcasted_iota(jnp.int32, (TBLK, 11), 1)
    oh_s = (slk_ref[...][:, None] == iot_s).astype(jnp.float32)
    oh_p = (plk_ref[...][:, None] == iot_p).astype(jnp.float32)
    out_ref[...] = (main
                    + jnp.dot(oh_s, sproj, preferred_element_type=jnp.float32)
                    + jnp.dot(oh_p, pproj, preferred_element_type=jnp.float32)
                    + b_ref[...])


def _build_fused_table(phoneme_table, sutra_table, position_table,
                       sutra_lookup, position_lookup, W, b):
    grid = pl.cdiv(V, TBLK)
    return pl.pallas_call(
        _fused_table_body,
        grid=(grid,),
        in_specs=[
            pl.BlockSpec((TBLK, D), lambda i: (i, 0)),
            pl.BlockSpec((TBLK,), lambda i: (i,)),
            pl.BlockSpec((TBLK,), lambda i: (i,)),
            pl.BlockSpec((15, D), lambda i: (0, 0)),
            pl.BlockSpec((11, D), lambda i: (0, 0)),
            pl.BlockSpec((3 * D, D), lambda i: (0, 0)),
            pl.BlockSpec((1, D), lambda i: (0, 0)),
        ],
        out_specs=pl.BlockSpec((TBLK, D), lambda i: (i, 0)),
        out_shape=jax.ShapeDtypeStruct((V, D), jnp.float32),
    )(phoneme_table, sutra_lookup, position_lookup,
      sutra_table, position_table, W, b[None, :])


def _gather_rows(table, idx):
    mesh = plsc.VectorSubcoreMesh(core_axis_name="c", subcore_axis_name="s")

    @functools.partial(
        pl.kernel,
        out_type=jax.ShapeDtypeStruct((N, D), jnp.float32),
        mesh=mesh,
        scratch_types=[
            pltpu.VMEM((NCHUNK, CHUNK), jnp.int32),
            pltpu.VMEM((NBUF, CHUNK, D), jnp.float32),
        ] + [pltpu.SemaphoreType.DMA] * (2 * NBUF),
    )
    def gather_kernel(table_hbm, idx_hbm, out_hbm, idx_v, rows_v, *sems):
        gsem, osem = sems[:NBUF], sems[NBUF:]
        wid = lax.axis_index("s") * NC + lax.axis_index("c")
        base = wid * NPW
        pltpu.sync_copy(idx_hbm.at[wid], idx_v)

        def gather_start(j, bi):
            pltpu.async_copy(table_hbm.at[idx_v.at[j]], rows_v.at[bi],
                             gsem[bi])

        def gather_wait(j, bi):
            pltpu.make_async_copy(table_hbm.at[idx_v.at[j]], rows_v.at[bi],
                                  gsem[bi]).wait()

        def out_start(j, bi):
            pltpu.async_copy(rows_v.at[bi],
                             out_hbm.at[pl.ds(base + j * CHUNK, CHUNK)],
                             osem[bi])

        def out_wait(j, bi):
            pltpu.make_async_copy(
                rows_v.at[bi], out_hbm.at[pl.ds(base + j * CHUNK, CHUNK)],
                osem[bi]).wait()

        for bi in range(NBUF):
            gather_start(bi, bi)
        gather_wait(0, 0)
        out_start(0, 0)

        def steady(j0):
            for dj in range(NBUF):
                j = j0 + dj
                nb = dj
                out_wait(j - 1, nb)
                gather_start(j + NBUF - 1, nb)
                bi = (dj + 1) % NBUF
                gather_wait(j, bi)
                out_start(j, bi)

        pl.loop(1, NCHUNK - NBUF + 1, step=NBUF)(steady)
        for j in range(NCHUNK - NBUF + 1, NCHUNK):
            bi = j % NBUF
            gather_wait(j, bi)
            out_start(j, bi)
        for j in range(NCHUNK - NBUF, NCHUNK):
            out_wait(j, j % NBUF)

    return gather_kernel(table, idx)


def kernel(phoneme_indices, phoneme_table, sutra_table, position_table,
           sutra_lookup, position_lookup, W, b):
    fused = _build_fused_table(phoneme_table, sutra_table, position_table,
                               sutra_lookup, position_lookup, W, b)
    idx = phoneme_indices.reshape(NW, NCHUNK, CHUNK).astype(jnp.int32)
    out = _gather_rows(fused, idx)
    return out.reshape(B, S, D)


# speedup vs baseline: 40.5315x; 1.0072x over previous
import functools

import jax
import jax.numpy as jnp
from jax import lax
from jax.experimental import pallas as pl
from jax.experimental.pallas import tpu as pltpu
from jax.experimental.pallas import tpu_sc as plsc

B, S, V, D = 1024, 200, 100000, 128
N = B * S

NC, NS = 2, 16
NW = NC * NS
NPW = N // NW
CHUNK = 128
NCHUNK = NPW // CHUNK
NBUF = 5

TBLK = 16384


def _fused_table_body(pt_ref, slk_ref, plk_ref, st_ref, po_ref, w_ref, b_ref,
                      out_ref):
    w0 = w_ref[0:D, :]
    w1 = w_ref[D:2 * D, :]
    w2 = w_ref[2 * D:3 * D, :]
    sproj = jnp.dot(st_ref[...], w1, preferred_element_type=jnp.float32)
    pproj = jnp.dot(po_ref[...], w2, preferred_element_type=jnp.float32)
    main = jnp.dot(pt_ref[...], w0, preferred_element_type=jnp.float32)
    iot_s = lax.broadcasted_iota(jnp.int32, (TBLK, 15), 1)
    iot_p = lax.broadcasted_iota(jnp.int32, (TBLK, 11), 1)
    oh_s = (slk_ref[...][:, None] == iot_s).astype(jnp.float32)
    oh_p = (plk_ref[...][:, None] == iot_p).astype(jnp.float32)
    out_ref[...] = (main
                    + jnp.dot(oh_s, sproj, preferred_element_type=jnp.float32)
                    + jnp.dot(oh_p, pproj, preferred_element_type=jnp.float32)
                    + b_ref[...])


def _build_fused_table(phoneme_table, sutra_table, position_table,
                       sutra_lookup, position_lookup, W, b):
    grid = pl.cdiv(V, TBLK)
    return pl.pallas_call(
        _fused_table_body,
        grid=(grid,),
        in_specs=[
            pl.BlockSpec((TBLK, D), lambda i: (i, 0)),
            pl.BlockSpec((TBLK,), lambda i: (i,)),
            pl.BlockSpec((TBLK,), lambda i: (i,)),
            pl.BlockSpec((15, D), lambda i: (0, 0)),
            pl.BlockSpec((11, D), lambda i: (0, 0)),
            pl.BlockSpec((3 * D, D), lambda i: (0, 0)),
            pl.BlockSpec((1, D), lambda i: (0, 0)),
        ],
        out_specs=pl.BlockSpec((TBLK, D), lambda i: (i, 0)),
        out_shape=jax.ShapeDtypeStruct((V, D), jnp.float32),
    )(phoneme_table, sutra_lookup, position_lookup,
      sutra_table, position_table, W, b[None, :])


def _gather_rows(table, idx):
    mesh = plsc.VectorSubcoreMesh(core_axis_name="c", subcore_axis_name="s")

    @functools.partial(
        pl.kernel,
        out_type=jax.ShapeDtypeStruct((N, D), jnp.float32),
        mesh=mesh,
        scratch_types=[
            pltpu.VMEM((NCHUNK, CHUNK), jnp.int32),
            pltpu.VMEM((NBUF, CHUNK, D), jnp.float32),
        ] + [pltpu.SemaphoreType.DMA] * (2 * NBUF),
    )
    def gather_kernel(table_hbm, idx_hbm, out_hbm, idx_v, rows_v, *sems):
        gsem, osem = sems[:NBUF], sems[NBUF:]
        wid = lax.axis_index("s") * NC + lax.axis_index("c")
        base = wid * NPW
        pltpu.sync_copy(idx_hbm.at[wid], idx_v)

        def gather_start(j, bi):
            pltpu.async_copy(table_hbm.at[idx_v.at[j]], rows_v.at[bi],
                             gsem[bi])

        def gather_wait(j, bi):
            pltpu.make_async_copy(table_hbm.at[idx_v.at[j]], rows_v.at[bi],
                                  gsem[bi]).wait()

        def out_start(j, bi):
            pltpu.async_copy(rows_v.at[bi],
                             out_hbm.at[pl.ds(base + j * CHUNK, CHUNK)],
                             osem[bi])

        def out_wait(j, bi):
            pltpu.make_async_copy(
                rows_v.at[bi], out_hbm.at[pl.ds(base + j * CHUNK, CHUNK)],
                osem[bi]).wait()

        for bi in range(NBUF):
            gather_start(bi, bi)
        gather_wait(0, 0)
        out_start(0, 0)

        def steady(j0):
            for dj in range(NBUF):
                j = j0 + dj
                nb = dj
                out_wait(j - 1, nb)
                gather_start(j + NBUF - 1, nb)
                bi = (dj + 1) % NBUF
                gather_wait(j, bi)
                out_start(j, bi)

        pl.loop(1, NCHUNK - NBUF + 1, step=NBUF)(steady)
        for j in range(NCHUNK - NBUF + 1, NCHUNK):
            bi = j % NBUF
            gather_wait(j, bi)
            out_start(j, bi)
        for j in range(NCHUNK - NBUF, NCHUNK):
            out_wait(j, j % NBUF)

    return gather_kernel(table, idx)


def kernel(phoneme_indices, phoneme_table, sutra_table, position_table,
           sutra_lookup, position_lookup, W, b):
    fused = _build_fused_table(phoneme_table, sutra_table, position_table,
                               sutra_lookup, position_lookup, W, b)
    idx = phoneme_indices.reshape(NW, NCHUNK, CHUNK).astype(jnp.int32)
    out = _gather_rows(fused, idx)
    return out.reshape(B, S, D)


# speedup vs baseline: 40.6520x; 1.0030x over previous
import functools

import jax
import jax.numpy as jnp
from jax import lax
from jax.experimental import pallas as pl
from jax.experimental.pallas import tpu as pltpu
from jax.experimental.pallas import tpu_sc as plsc

B, S, V, D = 1024, 200, 100000, 128
N = B * S

NC, NS = 2, 16
NW = NC * NS
NPW = N // NW
CHUNK = 128
NCHUNK = NPW // CHUNK
NBUF = 5

TBLK = 12288


def _fused_table_body(pt_ref, slk_ref, plk_ref, st_ref, po_ref, w_ref, b_ref,
                      out_ref):
    w0 = w_ref[0:D, :]
    w1 = w_ref[D:2 * D, :]
    w2 = w_ref[2 * D:3 * D, :]
    sproj = jnp.dot(st_ref[...], w1, preferred_element_type=jnp.float32)
    pproj = jnp.dot(po_ref[...], w2, preferred_element_type=jnp.float32)
    main = jnp.dot(pt_ref[...], w0, preferred_element_type=jnp.float32)
    iot_s = lax.broadcasted_iota(jnp.int32, (TBLK, 15), 1)
    iot_p = lax.broadcasted_iota(jnp.int32, (TBLK, 11), 1)
    oh_s = (slk_ref[...][:, None] == iot_s).astype(jnp.float32)
    oh_p = (plk_ref[...][:, None] == iot_p).astype(jnp.float32)
    out_ref[...] = (main
                    + jnp.dot(oh_s, sproj, preferred_element_type=jnp.float32)
                    + jnp.dot(oh_p, pproj, preferred_element_type=jnp.float32)
                    + b_ref[...])


def _build_fused_table(phoneme_table, sutra_table, position_table,
                       sutra_lookup, position_lookup, W, b):
    grid = pl.cdiv(V, TBLK)
    return pl.pallas_call(
        _fused_table_body,
        grid=(grid,),
        in_specs=[
            pl.BlockSpec((TBLK, D), lambda i: (i, 0)),
            pl.BlockSpec((TBLK,), lambda i: (i,)),
            pl.BlockSpec((TBLK,), lambda i: (i,)),
            pl.BlockSpec((15, D), lambda i: (0, 0)),
            pl.BlockSpec((11, D), lambda i: (0, 0)),
            pl.BlockSpec((3 * D, D), lambda i: (0, 0)),
            pl.BlockSpec((1, D), lambda i: (0, 0)),
        ],
        out_specs=pl.BlockSpec((TBLK, D), lambda i: (i, 0)),
        out_shape=jax.ShapeDtypeStruct((V, D), jnp.float32),
    )(phoneme_table, sutra_lookup, position_lookup,
      sutra_table, position_table, W, b[None, :])


def _gather_rows(table, idx):
    mesh = plsc.VectorSubcoreMesh(core_axis_name="c", subcore_axis_name="s")

    @functools.partial(
        pl.kernel,
        out_type=jax.ShapeDtypeStruct((N, D), jnp.float32),
        mesh=mesh,
        scratch_types=[
            pltpu.VMEM((NCHUNK, CHUNK), jnp.int32),
            pltpu.VMEM((NBUF, CHUNK, D), jnp.float32),
        ] + [pltpu.SemaphoreType.DMA] * (2 * NBUF),
    )
    def gather_kernel(table_hbm, idx_hbm, out_hbm, idx_v, rows_v, *sems):
        gsem, osem = sems[:NBUF], sems[NBUF:]
        wid = lax.axis_index("s") * NC + lax.axis_index("c")
        base = wid * NPW
        pltpu.sync_copy(idx_hbm.at[wid], idx_v)

        def gather_start(j, bi):
            pltpu.async_copy(table_hbm.at[idx_v.at[j]], rows_v.at[bi],
                             gsem[bi])

        def gather_wait(j, bi):
            pltpu.make_async_copy(table_hbm.at[idx_v.at[j]], rows_v.at[bi],
                                  gsem[bi]).wait()

        def out_start(j, bi):
            pltpu.async_copy(rows_v.at[bi],
                             out_hbm.at[pl.ds(base + j * CHUNK, CHUNK)],
                             osem[bi])

        def out_wait(j, bi):
            pltpu.make_async_copy(
                rows_v.at[bi], out_hbm.at[pl.ds(base + j * CHUNK, CHUNK)],
                osem[bi]).wait()

        for bi in range(NBUF):
            gather_start(bi, bi)
        gather_wait(0, 0)
        out_start(0, 0)

        def steady(j0):
            for dj in range(NBUF):
                j = j0 + dj
                nb = dj
                out_wait(j - 1, nb)
                gather_start(j + NBUF - 1, nb)
                bi = (dj + 1) % NBUF
                gather_wait(j, bi)
                out_start(j, bi)

        pl.loop(1, NCHUNK - NBUF + 1, step=NBUF)(steady)
        for j in range(NCHUNK - NBUF + 1, NCHUNK):
            bi = j % NBUF
            gather_wait(j, bi)
            out_start(j, bi)
        for j in range(NCHUNK - NBUF, NCHUNK):
            out_wait(j, j % NBUF)

    return gather_kernel(table, idx)


def kernel(phoneme_indices, phoneme_table, sutra_table, position_table,
           sutra_lookup, position_lookup, W, b):
    fused = _build_fused_table(phoneme_table, sutra_table, position_table,
                               sutra_lookup, position_lookup, W, b)
    idx = phoneme_indices.reshape(NW, NCHUNK, CHUNK).astype(jnp.int32)
    out = _gather_rows(fused, idx)
    return out.reshape(B, S, D)


# speedup vs baseline: 40.6582x; 1.0002x over previous
import functools

import jax
import jax.numpy as jnp
from jax import lax
from jax.experimental import pallas as pl
from jax.experimental.pallas import tpu as pltpu
from jax.experimental.pallas import tpu_sc as plsc

B, S, V, D = 1024, 200, 100000, 128
N = B * S

NC, NS = 2, 16
NW = NC * NS
NPW = N // NW
CHUNK = 128
NCHUNK = NPW // CHUNK
NBUF = 5

TBLK = 20480


def _fused_table_body(pt_ref, slk_ref, plk_ref, st_ref, po_ref, w_ref, b_ref,
                      out_ref):
    w0 = w_ref[0:D, :]
    w1 = w_ref[D:2 * D, :]
    w2 = w_ref[2 * D:3 * D, :]
    sproj = jnp.dot(st_ref[...], w1, preferred_element_type=jnp.float32)
    pproj = jnp.dot(po_ref[...], w2, preferred_element_type=jnp.float32)
    main = jnp.dot(pt_ref[...], w0, preferred_element_type=jnp.float32)
    iot_s = lax.broadcasted_iota(jnp.int32, (TBLK, 15), 1)
    iot_p = lax.broadcasted_iota(jnp.int32, (TBLK, 11), 1)
    oh_s = (slk_ref[...][:, None] == iot_s).astype(jnp.float32)
    oh_p = (plk_ref[...][:, None] == iot_p).astype(jnp.float32)
    out_ref[...] = (main
                    + jnp.dot(oh_s, sproj, preferred_element_type=jnp.float32)
                    + jnp.dot(oh_p, pproj, preferred_element_type=jnp.float32)
                    + b_ref[...])


def _build_fused_table(phoneme_table, sutra_table, position_table,
                       sutra_lookup, position_lookup, W, b):
    grid = pl.cdiv(V, TBLK)
    return pl.pallas_call(
        _fused_table_body,
        grid=(grid,),
        in_specs=[
            pl.BlockSpec((TBLK, D), lambda i: (i, 0)),
            pl.BlockSpec((TBLK,), lambda i: (i,)),
            pl.BlockSpec((TBLK,), lambda i: (i,)),
            pl.BlockSpec((15, D), lambda i: (0, 0)),
            pl.BlockSpec((11, D), lambda i: (0, 0)),
            pl.BlockSpec((3 * D, D), lambda i: (0, 0)),
            pl.BlockSpec((1, D), lambda i: (0, 0)),
        ],
        out_specs=pl.BlockSpec((TBLK, D), lambda i: (i, 0)),
        out_shape=jax.ShapeDtypeStruct((V, D), jnp.float32),
    )(phoneme_table, sutra_lookup, position_lookup,
      sutra_table, position_table, W, b[None, :])


def _gather_rows(table, idx):
    mesh = plsc.VectorSubcoreMesh(core_axis_name="c", subcore_axis_name="s")

    @functools.partial(
        pl.kernel,
        out_type=jax.ShapeDtypeStruct((N, D), jnp.float32),
        mesh=mesh,
        scratch_types=[
            pltpu.VMEM((NCHUNK, CHUNK), jnp.int32),
            pltpu.VMEM((NBUF, CHUNK, D), jnp.float32),
        ] + [pltpu.SemaphoreType.DMA] * (2 * NBUF),
    )
    def gather_kernel(table_hbm, idx_hbm, out_hbm, idx_v, rows_v, *sems):
        gsem, osem = sems[:NBUF], sems[NBUF:]
        wid = lax.axis_index("s") * NC + lax.axis_index("c")
        base = wid * NPW
        pltpu.sync_copy(idx_hbm.at[wid], idx_v)

        def gather_start(j, bi):
            pltpu.async_copy(table_hbm.at[idx_v.at[j]], rows_v.at[bi],
                             gsem[bi])

        def gather_wait(j, bi):
            pltpu.make_async_copy(table_hbm.at[idx_v.at[j]], rows_v.at[bi],
                                  gsem[bi]).wait()

        def out_start(j, bi):
            pltpu.async_copy(rows_v.at[bi],
                             out_hbm.at[pl.ds(base + j * CHUNK, CHUNK)],
                             osem[bi])

        def out_wait(j, bi):
            pltpu.make_async_copy(
                rows_v.at[bi], out_hbm.at[pl.ds(base + j * CHUNK, CHUNK)],
                osem[bi]).wait()

        for bi in range(NBUF):
            gather_start(bi, bi)
        gather_wait(0, 0)
        out_start(0, 0)

        def steady(j0):
            for dj in range(NBUF):
                j = j0 + dj
                nb = dj
                out_wait(j - 1, nb)
                gather_start(j + NBUF - 1, nb)
                bi = (dj + 1) % NBUF
                gather_wait(j, bi)
                out_start(j, bi)

        pl.loop(1, NCHUNK - NBUF + 1, step=NBUF)(steady)
        for j in range(NCHUNK - NBUF + 1, NCHUNK):
            bi = j % NBUF
            gather_wait(j, bi)
            out_start(j, bi)
        for j in range(NCHUNK - NBUF, NCHUNK):
            out_wait(j, j % NBUF)

    return gather_kernel(table, idx)


def kernel(phoneme_indices, phoneme_table, sutra_table, position_table,
           sutra_lookup, position_lookup, W, b):
    fused = _build_fused_table(phoneme_table, sutra_table, position_table,
                               sutra_lookup, position_lookup, W, b)
    idx = phoneme_indices.reshape(NW, NCHUNK, CHUNK).astype(jnp.int32)
    out = _gather_rows(fused, idx)
    return out.reshape(B, S, D)
